# Initial kernel scaffold; baseline (speedup 1.0000x reference)
import jax
import jax.numpy as jnp
from jax.experimental import pallas as pl


def kernel(preds, labels):
    raise NotImplementedError("write your pallas kernel here")


# speedup vs baseline: 1.6065x; 1.6065x over previous
import functools

import jax
import jax.numpy as jnp
from jax import lax
from jax.experimental import pallas as pl
from jax.experimental.pallas import tpu as pltpu
from jax.experimental.pallas import tpu_sc as plsc

NC, NS, L = 2, 16, 16
NW = NC * NS
B, N, K = 128, 100000, 32
RPW = B // NW
CHUNK_V = 25
CHUNK = CHUNK_V * L
NCHUNK = N // CHUNK
NEG_INF = float("-inf")


def _merge(T1, T2, v):
    vs = jnp.sort(v)
    rvs = lax.rev(vs, (0,))
    p = jnp.sort(jnp.maximum(T2, rvs))
    rp = lax.rev(p, (0,))
    T1n = jnp.sort(jnp.maximum(T1, rp))
    T2n = jnp.sort(jnp.minimum(T1, rp))
    return T1n, T2n


def _row_topk(row_v):
    T1 = jnp.full((L,), NEG_INF, jnp.float32)
    T2 = jnp.full((L,), NEG_INF, jnp.float32)
    tq = jnp.full((L,), NEG_INF, jnp.float32)

    def chunk_body(c, carry):
        T1, T2, tq = carry
        base = c * CHUNK
        acc = row_v[pl.ds(base, L)]
        for k in range(1, CHUNK_V):
            acc = jnp.maximum(acc, row_v[pl.ds(base + k * L, L)])

        def rescan(carry):
            T1, T2, tq = carry

            def vec_body(k, carry):
                T1, T2, tq = carry
                v = row_v[pl.ds(base + k * L, L)]

                def do_merge(carry):
                    T1, T2, _ = carry
                    T1n, T2n = _merge(T1, T2, v)
                    tqn = jnp.full((L,), jnp.min(T2n))
                    return (T1n, T2n, tqn)

                return lax.cond(jnp.any(v > tq), do_merge, lambda c: c,
                                (T1, T2, tq))

            return lax.fori_loop(0, CHUNK_V, vec_body, (T1, T2, tq))

        return lax.cond(jnp.any(acc > tq), rescan, lambda c: c, (T1, T2, tq))

    return lax.fori_loop(0, NCHUNK, chunk_body, (T1, T2, tq))


_LANE0 = None


def _store_scalar(stage_v, idx, val_splat):
    mask = lax.iota(jnp.int32, L) == 0
    idxv = jnp.full((L,), idx, jnp.int32)
    plsc.store_scatter(stage_v, [idxv], val_splat, mask=mask)


def _sc_kernel(preds_hbm, labels_hbm, out_hbm, row_v, lab_v, stage_v):
    wid = lax.axis_index("s") * NC + lax.axis_index("c")
    pltpu.sync_copy(labels_hbm, lab_v)
    for j in range(RPW):
        r = wid * RPW + j
        pltpu.sync_copy(preds_hbm.at[r], row_v)
        T1, T2, _ = _row_topk(row_v)
        m = jnp.max(T1)
        msplat = jnp.full((L,), m)
        s = jnp.sum(jnp.exp(T1 - msplat)) + jnp.sum(jnp.exp(T2 - msplat))
        lab_splat = plsc.load_gather(
            lab_v, [jnp.full((L,), r, jnp.int32)])
        neg_splat = plsc.load_gather(row_v, [lab_splat])
        _store_scalar(stage_v, j, msplat)
        _store_scalar(stage_v, RPW + j, jnp.full((L,), s))
        _store_scalar(stage_v, 2 * RPW + j, neg_splat)
    pltpu.sync_copy(stage_v, out_hbm.at[wid])


@functools.partial(jax.jit, static_argnames=())
def _sc_stage(preds, labels32):
    mesh = plsc.VectorSubcoreMesh(core_axis_name="c", subcore_axis_name="s",
                                  num_cores=NC, num_subcores=NS)
    f = pl.kernel(
        _sc_kernel,
        out_type=jax.ShapeDtypeStruct((NW, 3 * RPW), jnp.float32),
        mesh=mesh,
        scratch_types=[
            pltpu.VMEM((N,), jnp.float32),
            pltpu.VMEM((B,), jnp.int32),
            pltpu.VMEM((3 * RPW,), jnp.float32),
        ],
        compiler_params=pltpu.CompilerParams(needs_layout_passes=False),
    )
    return f(preds, labels32)


def _tc_epilogue_kernel(x_ref, o_ref):
    x = x_ref[...]
    m = x[:, 0:RPW]
    s = x[:, RPW:2 * RPW]
    neg = x[:, 2 * RPW:3 * RPW]
    loss = m + jnp.log(s) - neg
    o_ref[0, 0] = jnp.mean(loss)


def kernel(preds, labels):
    labels32 = labels.astype(jnp.int32)
    stats = _sc_stage(preds, labels32)
    out = pl.pallas_call(
        _tc_epilogue_kernel,
        out_shape=jax.ShapeDtypeStruct((1, 1), jnp.float32),
        out_specs=pl.BlockSpec(memory_space=pltpu.SMEM),
    )(stats)
    return out.reshape(())


# speedup vs baseline: 2.6777x; 1.6668x over previous
import functools

import jax
import jax.numpy as jnp
from jax import lax
from jax.experimental import pallas as pl
from jax.experimental.pallas import tpu as pltpu
from jax.experimental.pallas import tpu_sc as plsc

NC, NS, L = 2, 16, 16
NW = NC * NS
B, N, K = 128, 100000, 32
RPW = B // NW
CHUNK_V = 25
CHUNK = CHUNK_V * L
NCHUNK = N // CHUNK
NEG_INF = float("-inf")


def _merge(T1, T2, v):
    vs = jnp.sort(v)
    rvs = lax.rev(vs, (0,))
    p = jnp.sort(jnp.maximum(T2, rvs))
    rp = lax.rev(p, (0,))
    T1n = jnp.sort(jnp.maximum(T1, rp))
    T2n = jnp.sort(jnp.minimum(T1, rp))
    return T1n, T2n


def _row_topk(row_v, cmv_v):
    ninf = jnp.full((L,), NEG_INF, jnp.float32)

    @plsc.parallel_loop(0, NCHUNK, carry=(ninf, ninf))
    def passA(c, carry):
        m1, m2 = carry
        base = c * CHUNK
        acc = row_v[pl.ds(base, L)]
        for k in range(1, CHUNK_V):
            acc = jnp.maximum(acc, row_v[pl.ds(base + k * L, L)])
        cmv_v[pl.ds(c * L, L)] = acc
        m2n = jnp.maximum(m2, jnp.minimum(m1, acc))
        m1n = jnp.maximum(m1, acc)
        return (m1n, m2n)

    _, m2 = passA
    t0q = jnp.full((L,), jnp.min(m2))

    def phaseB(c, carry):
        acc = cmv_v[pl.ds(c * L, L)]

        def rescan(carry):
            def vec_body(k, carry):
                v = row_v[pl.ds(c * CHUNK + k * L, L)]

                def do_merge(carry):
                    T1, T2 = carry
                    return _merge(T1, T2, v)

                return lax.cond(jnp.any(v >= t0q), do_merge, lambda q: q,
                                carry)

            return lax.fori_loop(0, CHUNK_V, vec_body, carry)

        return lax.cond(jnp.any(acc >= t0q), rescan, lambda q: q, carry)

    return lax.fori_loop(0, NCHUNK, phaseB, (ninf, ninf))


_LANE0 = None


def _store_scalar(stage_v, idx, val_splat):
    mask = lax.iota(jnp.int32, L) == 0
    idxv = jnp.full((L,), idx, jnp.int32)
    plsc.store_scatter(stage_v, [idxv], val_splat, mask=mask)


def _sc_kernel(preds_hbm, labels_hbm, out_hbm, row_v, lab_v, stage_v, cmv_v):
    wid = lax.axis_index("s") * NC + lax.axis_index("c")
    pltpu.sync_copy(labels_hbm, lab_v)
    for j in range(RPW):
        r = wid * RPW + j
        pltpu.sync_copy(preds_hbm.at[r], row_v)
        T1, T2 = _row_topk(row_v, cmv_v)
        m = jnp.max(T1)
        msplat = jnp.full((L,), m)
        s = jnp.sum(jnp.exp(T1 - msplat)) + jnp.sum(jnp.exp(T2 - msplat))
        lab_splat = plsc.load_gather(
            lab_v, [jnp.full((L,), r, jnp.int32)])
        neg_splat = plsc.load_gather(row_v, [lab_splat])
        _store_scalar(stage_v, j, msplat)
        _store_scalar(stage_v, RPW + j, jnp.full((L,), s))
        _store_scalar(stage_v, 2 * RPW + j, neg_splat)
    pltpu.sync_copy(stage_v, out_hbm.at[wid])


@functools.partial(jax.jit, static_argnames=())
def _sc_stage(preds, labels32):
    mesh = plsc.VectorSubcoreMesh(core_axis_name="c", subcore_axis_name="s",
                                  num_cores=NC, num_subcores=NS)
    f = pl.kernel(
        _sc_kernel,
        out_type=jax.ShapeDtypeStruct((NW, 3 * RPW), jnp.float32),
        mesh=mesh,
        scratch_types=[
            pltpu.VMEM((N,), jnp.float32),
            pltpu.VMEM((B,), jnp.int32),
            pltpu.VMEM((3 * RPW,), jnp.float32),
            pltpu.VMEM((NCHUNK * L,), jnp.float32),
        ],
        compiler_params=pltpu.CompilerParams(needs_layout_passes=False),
    )
    return f(preds, labels32)


def _tc_epilogue_kernel(x_ref, o_ref):
    x = x_ref[...]
    m = x[:, 0:RPW]
    s = x[:, RPW:2 * RPW]
    neg = x[:, 2 * RPW:3 * RPW]
    loss = m + jnp.log(s) - neg
    o_ref[0, 0] = jnp.mean(loss)


def kernel(preds, labels):
    labels32 = labels.astype(jnp.int32)
    stats = _sc_stage(preds, labels32)
    out = pl.pallas_call(
        _tc_epilogue_kernel,
        out_shape=jax.ShapeDtypeStruct((1, 1), jnp.float32),
        out_specs=pl.BlockSpec(memory_space=pltpu.SMEM),
    )(stats)
    return out.reshape(())


# speedup vs baseline: 4.8755x; 1.8208x over previous
import functools

import jax
import jax.numpy as jnp
from jax import lax
from jax.experimental import pallas as pl
from jax.experimental.pallas import tpu as pltpu
from jax.experimental.pallas import tpu_sc as plsc

NC, NS, L = 2, 16, 16
NW = NC * NS
B, N, K = 128, 100000, 32
RPW = B // NW
CHUNK_V = 25
CHUNK = CHUNK_V * L
NCHUNK = N // CHUNK
NEG_INF = float("-inf")


def _merge(T1, T2, v):
    vs = jnp.sort(v)
    rvs = lax.rev(vs, (0,))
    p = jnp.sort(jnp.maximum(T2, rvs))
    rp = lax.rev(p, (0,))
    T1n = jnp.sort(jnp.maximum(T1, rp))
    T2n = jnp.sort(jnp.minimum(T1, rp))
    return T1n, T2n


CAPL = 392
FLUSH_AT = CAPL - 2 * CHUNK_V


def _row_topk(row_v, cm_s, cand_v):
    ninf = jnp.full((L,), NEG_INF, jnp.float32)

    @plsc.parallel_loop(0, NCHUNK, carry=(ninf, ninf))
    def passA(c, carry):
        m1, m2 = carry
        base = c * CHUNK
        acc = row_v[pl.ds(base, L)]
        for k in range(1, CHUNK_V):
            acc = jnp.maximum(acc, row_v[pl.ds(base + k * L, L)])
        cm_s[c] = jnp.max(acc)
        m2n = jnp.maximum(m2, jnp.minimum(m1, acc))
        m1n = jnp.maximum(m1, acc)
        return (m1n, m2n)

    _, m2 = passA
    t0 = jnp.min(m2)
    t0q = jnp.full((L,), t0)
    iota = lax.iota(jnp.int32, L)
    zeros = jnp.zeros((L,), jnp.int32)

    def flush(T1, T2, olane):
        nmax = jnp.max(olane)

        def body(d, carry):
            T1, T2 = carry
            v = cand_v[pl.ds(d * L, L)]
            cand_v[pl.ds(d * L, L)] = ninf
            return _merge(T1, T2, v)

        T1, T2 = lax.fori_loop(0, nmax, body, (T1, T2))
        return T1, T2

    def phaseB(c, carry):
        def collect(carry):
            T1, T2, olane, wc = carry

            def maybe_flush(args):
                T1, T2, olane = args
                T1, T2 = flush(T1, T2, olane)
                return T1, T2, zeros

            T1, T2, olane = lax.cond(wc >= FLUSH_AT, maybe_flush,
                                     lambda a: a, (T1, T2, olane))
            wc = jnp.where(wc >= FLUSH_AT, CHUNK_V, wc + CHUNK_V)
            base = c * CHUNK
            for k in range(CHUNK_V):
                v = row_v[pl.ds(base + k * L, L)]
                mask = v >= t0q
                idx = lax.shift_left(olane, 4) + iota
                plsc.store_scatter(cand_v, [idx], v, mask=mask)
                olane = olane + jnp.where(mask, 1, 0)
            return T1, T2, olane, wc

        return lax.cond(cm_s[c] >= t0, collect, lambda q: q, carry)

    T1, T2, olane, _ = lax.fori_loop(
        0, NCHUNK, phaseB, (ninf, ninf, zeros, jnp.int32(0)))
    return flush(T1, T2, olane)


_LANE0 = None


def _store_scalar(stage_v, idx, val_splat):
    mask = lax.iota(jnp.int32, L) == 0
    idxv = jnp.full((L,), idx, jnp.int32)
    plsc.store_scatter(stage_v, [idxv], val_splat, mask=mask)


def _sc_kernel(preds_hbm, labels_hbm, out_hbm, row_v, lab_v, stage_v, cm_s,
               cand_v):
    wid = lax.axis_index("s") * NC + lax.axis_index("c")
    pltpu.sync_copy(labels_hbm, lab_v)
    ninf = jnp.full((L,), NEG_INF, jnp.float32)

    @plsc.parallel_loop(0, CAPL)
    def _(d):
        cand_v[pl.ds(d * L, L)] = ninf

    for j in range(RPW):
        r = wid * RPW + j
        pltpu.sync_copy(preds_hbm.at[r], row_v)
        T1, T2 = _row_topk(row_v, cm_s, cand_v)
        m = jnp.max(T1)
        msplat = jnp.full((L,), m)
        s = jnp.sum(jnp.exp(T1 - msplat)) + jnp.sum(jnp.exp(T2 - msplat))
        lab_splat = plsc.load_gather(
            lab_v, [jnp.full((L,), r, jnp.int32)])
        neg_splat = plsc.load_gather(row_v, [lab_splat])
        _store_scalar(stage_v, j, msplat)
        _store_scalar(stage_v, RPW + j, jnp.full((L,), s))
        _store_scalar(stage_v, 2 * RPW + j, neg_splat)
    pltpu.sync_copy(stage_v, out_hbm.at[wid])


@functools.partial(jax.jit, static_argnames=())
def _sc_stage(preds, labels32):
    mesh = plsc.VectorSubcoreMesh(core_axis_name="c", subcore_axis_name="s",
                                  num_cores=NC, num_subcores=NS)
    f = pl.kernel(
        _sc_kernel,
        out_type=jax.ShapeDtypeStruct((NW, 3 * RPW), jnp.float32),
        mesh=mesh,
        scratch_types=[
            pltpu.VMEM((N,), jnp.float32),
            pltpu.VMEM((B,), jnp.int32),
            pltpu.VMEM((3 * RPW,), jnp.float32),
            pltpu.SMEM((NCHUNK,), jnp.float32),
            pltpu.VMEM((CAPL * L,), jnp.float32),
        ],
        compiler_params=pltpu.CompilerParams(needs_layout_passes=False),
    )
    return f(preds, labels32)


def _tc_epilogue_kernel(x_ref, o_ref):
    x = x_ref[...]
    m = x[:, 0:RPW]
    s = x[:, RPW:2 * RPW]
    neg = x[:, 2 * RPW:3 * RPW]
    loss = m + jnp.log(s) - neg
    o_ref[0, 0] = jnp.mean(loss)


def kernel(preds, labels):
    labels32 = labels.astype(jnp.int32)
    stats = _sc_stage(preds, labels32)
    out = pl.pallas_call(
        _tc_epilogue_kernel,
        out_shape=jax.ShapeDtypeStruct((1, 1), jnp.float32),
        out_specs=pl.BlockSpec(memory_space=pltpu.SMEM),
    )(stats)
    return out.reshape(())
